# Initial kernel scaffold; baseline (speedup 1.0000x reference)
import jax
import jax.numpy as jnp
from jax.experimental import pallas as pl


def kernel(action_mask, embedding):
    raise NotImplementedError("write your pallas kernel here")


# speedup vs baseline: 4.6189x; 4.6189x over previous
import jax
import jax.numpy as jnp
from jax.experimental import pallas as pl

_BLOCK_B = 1024


def _bcast_body(emb_ref, out_ref):
    out_ref[...] = jnp.broadcast_to(emb_ref[...], out_ref.shape)


def kernel(action_mask, embedding):
    batch = action_mask.shape[0]
    num_types, dim = embedding.shape
    flat = embedding.reshape(1, num_types * dim)
    out = pl.pallas_call(
        _bcast_body,
        grid=(batch // _BLOCK_B,),
        in_specs=[pl.BlockSpec((1, num_types * dim), lambda i: (0, 0))],
        out_specs=pl.BlockSpec((_BLOCK_B, num_types * dim), lambda i: (i, 0)),
        out_shape=jax.ShapeDtypeStruct((batch, num_types * dim), embedding.dtype),
    )(flat)
    return out.reshape(batch, num_types, dim)
